# baseline (speedup 1.0000x reference)
import functools

import jax
import jax.numpy as jnp
from jax import lax
from jax.experimental import pallas as pl
from jax.experimental.pallas import tpu as pltpu
from jax.experimental.pallas import tpu_sc as plsc

_N = 50000
_NS = 8
_C = 128
_E = _N * _NS
_GB = 128
_NW = 32
_EP = 409600
_ROWS_W = _EP // _NW
_KB = _ROWS_W // _GB
_BN = 1000
_BK = _BN * _NS
_NBLK = _N // _BN

_PI = 3.141592653589793
_TWO_PI = 6.283185307179586


def _sc_gather(table, idx1d, d, tc_tiling=True, dtype=jnp.float32):
    mesh = plsc.VectorSubcoreMesh(core_axis_name="c", subcore_axis_name="s")

    @functools.partial(
        pl.kernel,
        mesh=mesh,
        compiler_params=pltpu.CompilerParams(use_tc_tiling_on_sc=tc_tiling),
        out_type=jax.ShapeDtypeStruct((_EP, d), dtype),
        scratch_types=[
            pltpu.VMEM((_ROWS_W,), jnp.int32),
            pltpu.VMEM((_GB, d), dtype),
            pltpu.VMEM((_GB, d), dtype),
            pltpu.SemaphoreType.DMA,
        ],
    )
    def k(table_hbm, idx_hbm, out_hbm, idx_v, rows0, rows1, sem):
        wid = lax.axis_index("s") * 2 + lax.axis_index("c")
        base = wid * _ROWS_W
        pltpu.sync_copy(idx_hbm.at[pl.ds(base, _ROWS_W)], idx_v)
        bufs = (rows0, rows1)
        pltpu.async_copy(table_hbm.at[idx_v.at[pl.ds(0, _GB)]], rows0, sem)
        pltpu.async_copy(table_hbm.at[idx_v.at[pl.ds(_GB, _GB)]], rows1, sem)

        def body(j2, carry):
            for b in range(2):
                j = j2 * 2 + b
                buf = bufs[b]
                pltpu.make_async_copy(
                    table_hbm.at[pl.ds(0, _GB)], buf, sem).wait()
                pltpu.sync_copy(buf, out_hbm.at[pl.ds(base + j * _GB, _GB)])

                @pl.when(j + 2 < _KB)
                def _():
                    pltpu.async_copy(
                        table_hbm.at[idx_v.at[pl.ds((j + 2) * _GB, _GB)]],
                        buf, sem)
            return carry

        lax.fori_loop(0, _KB // 2, body, 0)

    return k(table, idx1d)


def _dense_body(x_ref, p_ref, n_ref, wqk_ref, wkv_ref, bqk_ref,
                a_ref, kv_ref, g_ref):
    xb = x_ref[...]
    a_ref[...] = jnp.dot(xb, wqk_ref[...],
                         preferred_element_type=jnp.float32) + bqk_ref[...]
    kv_ref[...] = jnp.dot(
        xb, wkv_ref[...],
        preferred_element_type=jnp.float32).astype(jnp.bfloat16)
    pb = p_ref[...]
    nb = n_ref[...]
    px = pb[:, 0:1]
    py = pb[:, 1:2]
    r = jnp.sqrt(px * px + py * py + 1e-12)
    theta = jnp.arctan2(py, px)
    zpad = jnp.zeros((_BN, 8), jnp.float32)
    g_ref[...] = jnp.concatenate([pb, r, theta, nb, zpad], axis=1)


def _dense(x, p, n, wqk, wkv, bqk):
    return pl.pallas_call(
        _dense_body,
        grid=(_NBLK,),
        in_specs=[
            pl.BlockSpec((_BN, _C), lambda i: (i, 0)),
            pl.BlockSpec((_BN, 3), lambda i: (i, 0)),
            pl.BlockSpec((_BN, 3), lambda i: (i, 0)),
            pl.BlockSpec((_C, _C), lambda i: (0, 0)),
            pl.BlockSpec((_C, 2 * _C), lambda i: (0, 0)),
            pl.BlockSpec((1, _C), lambda i: (0, 0)),
        ],
        out_specs=[
            pl.BlockSpec((_BN, _C), lambda i: (i, 0)),
            pl.BlockSpec((_BN, 2 * _C), lambda i: (i, 0)),
            pl.BlockSpec((_BN, 16), lambda i: (i, 0)),
        ],
        out_shape=[
            jax.ShapeDtypeStruct((_N, _C), jnp.float32),
            jax.ShapeDtypeStruct((_N, 2 * _C), jnp.bfloat16),
            jax.ShapeDtypeStruct((_N, 16), jnp.float32),
        ],
    )(x, p, n, wqk, wkv, bqk)


def _edge_body(gg8_ref, gd_ref, tile16_ref, bp_ref, bn_ref, bt_ref, b23w_ref,
               w1r0t_ref, w1r1t_ref, eb1t_ref, cmask_ref,
               h8_ref, dwc_ref, hs_ref):
    i = pl.program_id(0)
    gg8 = gg8_ref[...]
    gdr = jnp.dot(gd_ref[...], tile16_ref[...],
                  preferred_element_type=jnp.float32)
    dd = gg8 - gdr
    s = dd * dd
    t = jnp.dot(s, bp_ref[...], preferred_element_type=jnp.float32) + 1e-12
    dist = jnp.sqrt(t)
    dwfull = jnp.exp(-dist)
    dwc_ref[...] = jnp.dot(dwfull, cmask_ref[...],
                           preferred_element_type=jnp.float32)
    sn = jnp.dot(s, bn_ref[...], preferred_element_type=jnp.float32) + 1e-12
    dn = jnp.sqrt(sn)
    th = jnp.dot(dd, bt_ref[...], preferred_element_type=jnp.float32)
    y = th + _PI
    mm = y - _TWO_PI * jnp.floor(y * (1.0 / _TWO_PI))
    dtheta = jnp.abs(mm - _PI)
    hzr = jnp.dot(jnp.abs(dd), b23w_ref[...],
                  preferred_element_type=jnp.float32)
    h = dn * w1r0t_ref[...] + dtheta * w1r1t_ref[...] + hzr + eb1t_ref[...]
    h8_ref[...] = h
    part = jnp.concatenate(
        [jnp.sum(h, axis=0, keepdims=True),
         jnp.sum(h * h, axis=0, keepdims=True)], axis=0)

    @pl.when(i == 0)
    def _():
        hs_ref[...] = jnp.zeros_like(hs_ref)

    hs_ref[...] += part


def _edge(gg8, g, tile16, bp, bnm, bt, b23w, w1r0t, w1r1t, eb1t, cmask):
    return pl.pallas_call(
        _edge_body,
        grid=(_NBLK,),
        in_specs=[
            pl.BlockSpec((_BN, _C), lambda i: (i, 0)),
            pl.BlockSpec((_BN, 16), lambda i: (i, 0)),
            pl.BlockSpec((16, _C), lambda i: (0, 0)),
            pl.BlockSpec((_C, _C), lambda i: (0, 0)),
            pl.BlockSpec((_C, _C), lambda i: (0, 0)),
            pl.BlockSpec((_C, _C), lambda i: (0, 0)),
            pl.BlockSpec((_C, _C), lambda i: (0, 0)),
            pl.BlockSpec((1, _C), lambda i: (0, 0)),
            pl.BlockSpec((1, _C), lambda i: (0, 0)),
            pl.BlockSpec((1, _C), lambda i: (0, 0)),
            pl.BlockSpec((_C, 8), lambda i: (0, 0)),
        ],
        out_specs=[
            pl.BlockSpec((_BN, _C), lambda i: (i, 0)),
            pl.BlockSpec((_BN, 8), lambda i: (i, 0)),
            pl.BlockSpec((2, _C), lambda i: (0, 0)),
        ],
        out_shape=[
            jax.ShapeDtypeStruct((_E // 8, _C), jnp.float32),
            jax.ShapeDtypeStruct((_N, 8), jnp.float32),
            jax.ShapeDtypeStruct((2, _C), jnp.float32),
        ],
    )(gg8, g, tile16, bp, bnm, bt, b23w, w1r0t, w1r1t, eb1t, cmask)


def _wstats_body(kg_ref, a_ref, h_ref, se_ref, sh_ref, ew2_ref, eb2_ref,
                 ws_ref):
    i = pl.program_id(0)
    hr = jnp.maximum(h_ref[...] * se_ref[...] + sh_ref[...], 0.0)
    emb = jnp.dot(hr, ew2_ref[...],
                  preferred_element_type=jnp.float32) + eb2_ref[...]
    a = a_ref[...]
    ar = jnp.broadcast_to(a[:, None, :], (_BN, _NS, _C)).reshape(_BK, _C)
    w = ar - kg_ref[...].astype(jnp.float32) + emb
    part = jnp.concatenate(
        [jnp.sum(w, axis=0, keepdims=True),
         jnp.sum(w * w, axis=0, keepdims=True)], axis=0)

    @pl.when(i == 0)
    def _():
        ws_ref[...] = jnp.zeros_like(ws_ref)

    ws_ref[...] += part


def _wstats(kg, a, h, se, sh, ew2, eb2r):
    return pl.pallas_call(
        _wstats_body,
        grid=(_NBLK,),
        in_specs=[
            pl.BlockSpec((_BK, _C), lambda i: (i, 0)),
            pl.BlockSpec((_BN, _C), lambda i: (i, 0)),
            pl.BlockSpec((_BK, 16), lambda i: (i, 0)),
            pl.BlockSpec((1, 16), lambda i: (0, 0)),
            pl.BlockSpec((1, 16), lambda i: (0, 0)),
            pl.BlockSpec((16, _C), lambda i: (0, 0)),
            pl.BlockSpec((1, _C), lambda i: (0, 0)),
        ],
        out_specs=pl.BlockSpec((2, _C), lambda i: (0, 0)),
        out_shape=jax.ShapeDtypeStruct((2, _C), jnp.float32),
    )(kg, a, h, se, sh, ew2, eb2r)


def _u_body(kg_ref, a_ref, h_ref, se_ref, sh_ref, ew2_ref, eb2_ref,
            s1_ref, t1_ref, lw1_ref, lbb1_ref, u_ref, us_ref):
    i = pl.program_id(0)
    hr = jnp.maximum(h_ref[...] * se_ref[...] + sh_ref[...], 0.0)
    emb = jnp.dot(hr, ew2_ref[...],
                  preferred_element_type=jnp.float32) + eb2_ref[...]
    a = a_ref[...]
    ar = jnp.broadcast_to(a[:, None, :], (_BN, _NS, _C)).reshape(_BK, _C)
    w = ar - kg_ref[...].astype(jnp.float32) + emb
    wn = jnp.maximum(w * s1_ref[...] + t1_ref[...], 0.0)
    u = jnp.dot(wn, lw1_ref[...],
                preferred_element_type=jnp.float32) + lbb1_ref[...]
    u_ref[...] = u
    part = jnp.concatenate(
        [jnp.sum(u, axis=0, keepdims=True),
         jnp.sum(u * u, axis=0, keepdims=True)], axis=0)

    @pl.when(i == 0)
    def _():
        us_ref[...] = jnp.zeros_like(us_ref)

    us_ref[...] += part


def _upass(kg, a, h, se, sh, ew2, eb2r, s1, t1, lw1, lbb1r):
    return pl.pallas_call(
        _u_body,
        grid=(_NBLK,),
        in_specs=[
            pl.BlockSpec((_BK, _C), lambda i: (i, 0)),
            pl.BlockSpec((_BN, _C), lambda i: (i, 0)),
            pl.BlockSpec((_BK, 16), lambda i: (i, 0)),
            pl.BlockSpec((1, 16), lambda i: (0, 0)),
            pl.BlockSpec((1, 16), lambda i: (0, 0)),
            pl.BlockSpec((16, _C), lambda i: (0, 0)),
            pl.BlockSpec((1, _C), lambda i: (0, 0)),
            pl.BlockSpec((1, _C), lambda i: (0, 0)),
            pl.BlockSpec((1, _C), lambda i: (0, 0)),
            pl.BlockSpec((_C, 16), lambda i: (0, 0)),
            pl.BlockSpec((1, 16), lambda i: (0, 0)),
        ],
        out_specs=[
            pl.BlockSpec((_BK, 16), lambda i: (i, 0)),
            pl.BlockSpec((2, 16), lambda i: (0, 0)),
        ],
        out_shape=[
            jax.ShapeDtypeStruct((_E, 16), jnp.float32),
            jax.ShapeDtypeStruct((2, 16), jnp.float32),
        ],
    )(kg, a, h, se, sh, ew2, eb2r, s1, t1, lw1, lbb1r)


def _final_body(u_ref, h_ref, vg_ref, vd_ref, dw_ref, s2_ref, t2_ref,
                lw2t_ref, lb2t_ref, se_ref, sh_ref, ew2_ref, eb2_ref, bv_ref,
                o_ref):
    un = jnp.maximum(u_ref[...] * s2_ref[...] + t2_ref[...], 0.0)
    tf = (jnp.dot(un, lw2t_ref[...], preferred_element_type=jnp.float32)
          + lb2t_ref[...]).reshape(_BN, _NS, _C)
    mx = jnp.max(tf, axis=1, keepdims=True)
    ex = jnp.exp(tf - mx)
    watt = ex / jnp.sum(ex, axis=1, keepdims=True)
    hr = jnp.maximum(h_ref[...] * se_ref[...] + sh_ref[...], 0.0)
    emb = jnp.dot(hr, ew2_ref[...],
                  preferred_element_type=jnp.float32) + eb2_ref[...]
    vd = vd_ref[...].astype(jnp.float32)
    vdr = jnp.broadcast_to(vd[:, None, :], (_BN, _NS, _C)).reshape(_BK, _C)
    xv = (vg_ref[...].astype(jnp.float32) - vdr
          + bv_ref[...]) * dw_ref[...]
    z = (xv + emb).reshape(_BN, _NS, _C)
    o_ref[...] = jnp.sum(z * watt, axis=1)


def _final(u, h, vg, v, dwf, s2, t2, lw2t, lb2t, se, sh, ew2, eb2r, bvr):
    return pl.pallas_call(
        _final_body,
        grid=(_NBLK,),
        in_specs=[
            pl.BlockSpec((_BK, 16), lambda i: (i, 0)),
            pl.BlockSpec((_BK, 16), lambda i: (i, 0)),
            pl.BlockSpec((_BK, _C), lambda i: (i, 1)),
            pl.BlockSpec((_BN, _C), lambda i: (i, 1)),
            pl.BlockSpec((_BK, 1), lambda i: (i, 0)),
            pl.BlockSpec((1, 16), lambda i: (0, 0)),
            pl.BlockSpec((1, 16), lambda i: (0, 0)),
            pl.BlockSpec((16, _C), lambda i: (0, 0)),
            pl.BlockSpec((1, _C), lambda i: (0, 0)),
            pl.BlockSpec((1, 16), lambda i: (0, 0)),
            pl.BlockSpec((1, 16), lambda i: (0, 0)),
            pl.BlockSpec((16, _C), lambda i: (0, 0)),
            pl.BlockSpec((1, _C), lambda i: (0, 0)),
            pl.BlockSpec((1, _C), lambda i: (0, 0)),
        ],
        out_specs=pl.BlockSpec((_BN, _C), lambda i: (i, 0)),
        out_shape=jax.ShapeDtypeStruct((_N, _C), jnp.float32),
    )(u, h, vg, v, dwf, s2, t2, lw2t, lb2t, se, sh, ew2, eb2r, bvr)


def _bn_affine(sums, g, b, count):
    mean = sums[0] / count
    var = sums[1] / count - mean * mean
    scale = g / jnp.sqrt(var + 1e-5)
    shift = b - mean * scale
    return scale.reshape(1, -1), shift.reshape(1, -1)


def kernel(p, n, x, idx, Wq, bq, Wk, bk, Wv, bv, eW1, eb1, eg1, ebe1, eW2,
           eb2, lg1, lb1, lW1, lbb1, lg2, lbe2, lW2, lb2):
    idxf = idx.reshape(-1).astype(jnp.int32)
    idx1d = jnp.concatenate([idxf, jnp.zeros((_EP - _E,), jnp.int32)])

    wqk = Wq + Wk
    bqk = (bq - bk).reshape(1, _C)
    wkv = jnp.concatenate([Wk, Wv], axis=1)
    a, kv, g = _dense(x, p, n, wqk, wkv, bqk)

    eye8 = jnp.eye(8, dtype=jnp.float32)
    tile16 = jnp.tile(jnp.eye(16, dtype=jnp.float32), (1, 8))
    blkp = jnp.zeros((16, 16), jnp.float32).at[0:3, :].set(1.0)
    blkn = jnp.zeros((16, 16), jnp.float32).at[5:8, :].set(1.0)
    blkt = jnp.zeros((16, 16), jnp.float32).at[4, :].set(1.0)
    blk23 = (jnp.zeros((16, 16), jnp.float32)
             .at[2, :].set(eW1[2]).at[3, :].set(eW1[3]))
    bp = jnp.kron(eye8, blkp)
    bnm = jnp.kron(eye8, blkn)
    bt = jnp.kron(eye8, blkt)
    b23w = jnp.kron(eye8, blk23)
    cmask = jnp.kron(eye8, jnp.zeros((16, 1), jnp.float32).at[0, 0].set(1.0))
    w1r0t = jnp.tile(eW1[0:1, :], (1, 8))
    w1r1t = jnp.tile(eW1[1:2, :], (1, 8))
    eb1t = jnp.tile(eb1.reshape(1, 16), (1, 8))

    gg = _sc_gather(g, idx1d, 16, tc_tiling=False)
    gg8 = gg.reshape(_EP // 8, _C)
    h8, dwc, hs128 = _edge(gg8, g, tile16, bp, bnm, bt, b23w, w1r0t, w1r1t,
                           eb1t, cmask)
    h = h8.reshape(_E, 16)
    dwf = dwc.reshape(_E, 1)
    hs = hs128.reshape(2, 8, 16).sum(axis=1)
    se, sh = _bn_affine(hs, eg1, ebe1, float(_E))

    kv_i32 = lax.bitcast_convert_type(kv.reshape(_N, _C, 2), jnp.int32)
    kvg_i32 = _sc_gather(kv_i32, idx1d, _C, dtype=jnp.int32)
    kvg = lax.bitcast_convert_type(
        kvg_i32, jnp.bfloat16).reshape(_EP, 2 * _C)
    eb2r = eb2.reshape(1, _C)
    ws = _wstats(kvg, a, h, se, sh, eW2, eb2r)
    s1, t1 = _bn_affine(ws, lg1, lb1, float(_E))

    u, us = _upass(kvg, a, h, se, sh, eW2, eb2r, s1, t1, lW1,
                   lbb1.reshape(1, 16))
    s2, t2 = _bn_affine(us, lg2, lbe2, float(_E))

    lw2t = jnp.dot(lW2, tile16)
    lb2t = jnp.tile(lb2.reshape(1, 16), (1, 8))
    out = _final(u, h, kvg, kv, dwf, s2, t2, lw2t, lb2t, se, sh,
                 eW2, eb2r, bv.reshape(1, _C))
    return out


# speedup vs baseline: 2.2809x; 2.2809x over previous
import functools

import jax
import jax.numpy as jnp
from jax import lax
from jax.experimental import pallas as pl
from jax.experimental.pallas import tpu as pltpu
from jax.experimental.pallas import tpu_sc as plsc

_N = 50000
_NS = 8
_C = 128
_E = _N * _NS
_GB = 128
_NW = 32
_EP = 409600
_ROWS_W = _EP // _NW
_KB = _ROWS_W // _GB
_BN = 1000
_BK = _BN * _NS
_NBLK = _N // _BN

_PI = 3.141592653589793
_TWO_PI = 6.283185307179586


def _sc_gather(table, idx1d, d, tc_tiling=True, dtype=jnp.float32):
    mesh = plsc.VectorSubcoreMesh(core_axis_name="c", subcore_axis_name="s")

    @functools.partial(
        pl.kernel,
        mesh=mesh,
        compiler_params=pltpu.CompilerParams(use_tc_tiling_on_sc=tc_tiling),
        out_type=jax.ShapeDtypeStruct((_EP, d), dtype),
        scratch_types=[
            pltpu.VMEM((_ROWS_W,), jnp.int32),
            pltpu.VMEM((_GB, d), dtype),
            pltpu.VMEM((_GB, d), dtype),
            pltpu.SemaphoreType.DMA,
        ],
    )
    def k(table_hbm, idx_hbm, out_hbm, idx_v, rows0, rows1, sem):
        wid = lax.axis_index("s") * 2 + lax.axis_index("c")
        base = wid * _ROWS_W
        pltpu.sync_copy(idx_hbm.at[pl.ds(base, _ROWS_W)], idx_v)
        bufs = (rows0, rows1)
        pltpu.async_copy(table_hbm.at[idx_v.at[pl.ds(0, _GB)]], rows0, sem)
        pltpu.async_copy(table_hbm.at[idx_v.at[pl.ds(_GB, _GB)]], rows1, sem)

        def body(j2, carry):
            for b in range(2):
                j = j2 * 2 + b
                buf = bufs[b]
                pltpu.make_async_copy(
                    table_hbm.at[pl.ds(0, _GB)], buf, sem).wait()
                pltpu.sync_copy(buf, out_hbm.at[pl.ds(base + j * _GB, _GB)])

                @pl.when(j + 2 < _KB)
                def _():
                    pltpu.async_copy(
                        table_hbm.at[idx_v.at[pl.ds((j + 2) * _GB, _GB)]],
                        buf, sem)
            return carry

        lax.fori_loop(0, _KB // 2, body, 0)

    return k(table, idx1d)


def _bf16_bits(f):
    b = lax.bitcast_convert_type(f, jnp.int32)
    return b + 0x7FFF + jnp.bitwise_and(jnp.right_shift(b, 16), 1)


def _dense_body(x_ref, p_ref, n_ref, wqk_ref, wk_ref, wv_ref, bqk_ref,
                a_ref, kv_ref, g_ref):
    xb = x_ref[...]
    a_ref[...] = jnp.dot(xb, wqk_ref[...],
                         preferred_element_type=jnp.float32) + bqk_ref[...]
    kb = _bf16_bits(jnp.dot(xb, wk_ref[...],
                            preferred_element_type=jnp.float32))
    vb = _bf16_bits(jnp.dot(xb, wv_ref[...],
                            preferred_element_type=jnp.float32))
    klo = jnp.bitwise_and(jnp.right_shift(kb, 16), 0xFFFF)
    vhi = jnp.bitwise_and(vb, jnp.int32(-65536))
    kv_ref[...] = jnp.bitwise_or(klo, vhi)
    pb = p_ref[...]
    nb = n_ref[...]
    px = pb[:, 0:1]
    py = pb[:, 1:2]
    r = jnp.sqrt(px * px + py * py + 1e-12)
    theta = jnp.arctan2(py, px)
    zpad = jnp.zeros((_BN, 8), jnp.float32)
    g_ref[...] = jnp.concatenate([pb, r, theta, nb, zpad], axis=1)


def _dense(x, p, n, wqk, wk, wv, bqk):
    return pl.pallas_call(
        _dense_body,
        grid=(_NBLK,),
        in_specs=[
            pl.BlockSpec((_BN, _C), lambda i: (i, 0)),
            pl.BlockSpec((_BN, 3), lambda i: (i, 0)),
            pl.BlockSpec((_BN, 3), lambda i: (i, 0)),
            pl.BlockSpec((_C, _C), lambda i: (0, 0)),
            pl.BlockSpec((_C, _C), lambda i: (0, 0)),
            pl.BlockSpec((_C, _C), lambda i: (0, 0)),
            pl.BlockSpec((1, _C), lambda i: (0, 0)),
        ],
        out_specs=[
            pl.BlockSpec((_BN, _C), lambda i: (i, 0)),
            pl.BlockSpec((_BN, _C), lambda i: (i, 0)),
            pl.BlockSpec((_BN, 16), lambda i: (i, 0)),
        ],
        out_shape=[
            jax.ShapeDtypeStruct((_N, _C), jnp.float32),
            jax.ShapeDtypeStruct((_N, _C), jnp.int32),
            jax.ShapeDtypeStruct((_N, 16), jnp.float32),
        ],
    )(x, p, n, wqk, wk, wv, bqk)


def _edge_body(gg8_ref, gd_ref, tile16_ref, bp_ref, bn_ref, bt_ref, b23w_ref,
               w1r0t_ref, w1r1t_ref, eb1t_ref, cmask_ref,
               h8_ref, dwc_ref, hs_ref):
    i = pl.program_id(0)
    gg8 = gg8_ref[...]
    gdr = jnp.dot(gd_ref[...], tile16_ref[...],
                  preferred_element_type=jnp.float32)
    dd = gg8 - gdr
    s = dd * dd
    t = jnp.dot(s, bp_ref[...], preferred_element_type=jnp.float32) + 1e-12
    dist = jnp.sqrt(t)
    dwfull = jnp.exp(-dist)
    dwc_ref[...] = jnp.dot(dwfull, cmask_ref[...],
                           preferred_element_type=jnp.float32)
    sn = jnp.dot(s, bn_ref[...], preferred_element_type=jnp.float32) + 1e-12
    dn = jnp.sqrt(sn)
    th = jnp.dot(dd, bt_ref[...], preferred_element_type=jnp.float32)
    y = th + _PI
    mm = y - _TWO_PI * jnp.floor(y * (1.0 / _TWO_PI))
    dtheta = jnp.abs(mm - _PI)
    hzr = jnp.dot(jnp.abs(dd), b23w_ref[...],
                  preferred_element_type=jnp.float32)
    h = dn * w1r0t_ref[...] + dtheta * w1r1t_ref[...] + hzr + eb1t_ref[...]
    h8_ref[...] = h
    part = jnp.concatenate(
        [jnp.sum(h, axis=0, keepdims=True),
         jnp.sum(h * h, axis=0, keepdims=True)], axis=0)

    @pl.when(i == 0)
    def _():
        hs_ref[...] = jnp.zeros_like(hs_ref)

    hs_ref[...] += part


def _edge(gg8, g, tile16, bp, bnm, bt, b23w, w1r0t, w1r1t, eb1t, cmask):
    return pl.pallas_call(
        _edge_body,
        grid=(_NBLK,),
        in_specs=[
            pl.BlockSpec((_BN, _C), lambda i: (i, 0)),
            pl.BlockSpec((_BN, 16), lambda i: (i, 0)),
            pl.BlockSpec((16, _C), lambda i: (0, 0)),
            pl.BlockSpec((_C, _C), lambda i: (0, 0)),
            pl.BlockSpec((_C, _C), lambda i: (0, 0)),
            pl.BlockSpec((_C, _C), lambda i: (0, 0)),
            pl.BlockSpec((_C, _C), lambda i: (0, 0)),
            pl.BlockSpec((1, _C), lambda i: (0, 0)),
            pl.BlockSpec((1, _C), lambda i: (0, 0)),
            pl.BlockSpec((1, _C), lambda i: (0, 0)),
            pl.BlockSpec((_C, 8), lambda i: (0, 0)),
        ],
        out_specs=[
            pl.BlockSpec((_BN, _C), lambda i: (i, 0)),
            pl.BlockSpec((_BN, 8), lambda i: (i, 0)),
            pl.BlockSpec((2, _C), lambda i: (0, 0)),
        ],
        out_shape=[
            jax.ShapeDtypeStruct((_E // 8, _C), jnp.float32),
            jax.ShapeDtypeStruct((_N, 8), jnp.float32),
            jax.ShapeDtypeStruct((2, _C), jnp.float32),
        ],
    )(gg8, g, tile16, bp, bnm, bt, b23w, w1r0t, w1r1t, eb1t, cmask)


def _wstats_body(kg_ref, a_ref, h_ref, se_ref, sh_ref, ew2_ref, eb2_ref,
                 ws_ref):
    i = pl.program_id(0)
    hr = jnp.maximum(h_ref[...] * se_ref[...] + sh_ref[...], 0.0)
    emb = jnp.dot(hr, ew2_ref[...],
                  preferred_element_type=jnp.float32) + eb2_ref[...]
    a = a_ref[...]
    ar = jnp.broadcast_to(a[:, None, :], (_BN, _NS, _C)).reshape(_BK, _C)
    kg = lax.bitcast_convert_type(
        jnp.left_shift(kg_ref[...], 16), jnp.float32)
    w = ar - kg + emb
    part = jnp.concatenate(
        [jnp.sum(w, axis=0, keepdims=True),
         jnp.sum(w * w, axis=0, keepdims=True)], axis=0)

    @pl.when(i == 0)
    def _():
        ws_ref[...] = jnp.zeros_like(ws_ref)

    ws_ref[...] += part


def _wstats(kg, a, h, se, sh, ew2, eb2r):
    return pl.pallas_call(
        _wstats_body,
        grid=(_NBLK,),
        in_specs=[
            pl.BlockSpec((_BK, _C), lambda i: (i, 0)),
            pl.BlockSpec((_BN, _C), lambda i: (i, 0)),
            pl.BlockSpec((_BK, 16), lambda i: (i, 0)),
            pl.BlockSpec((1, 16), lambda i: (0, 0)),
            pl.BlockSpec((1, 16), lambda i: (0, 0)),
            pl.BlockSpec((16, _C), lambda i: (0, 0)),
            pl.BlockSpec((1, _C), lambda i: (0, 0)),
        ],
        out_specs=pl.BlockSpec((2, _C), lambda i: (0, 0)),
        out_shape=jax.ShapeDtypeStruct((2, _C), jnp.float32),
    )(kg, a, h, se, sh, ew2, eb2r)


def _u_body(kg_ref, a_ref, h_ref, se_ref, sh_ref, ew2_ref, eb2_ref,
            s1_ref, t1_ref, lw1_ref, lbb1_ref, u_ref, us_ref):
    i = pl.program_id(0)
    hr = jnp.maximum(h_ref[...] * se_ref[...] + sh_ref[...], 0.0)
    emb = jnp.dot(hr, ew2_ref[...],
                  preferred_element_type=jnp.float32) + eb2_ref[...]
    a = a_ref[...]
    ar = jnp.broadcast_to(a[:, None, :], (_BN, _NS, _C)).reshape(_BK, _C)
    kg = lax.bitcast_convert_type(
        jnp.left_shift(kg_ref[...], 16), jnp.float32)
    w = ar - kg + emb
    wn = jnp.maximum(w * s1_ref[...] + t1_ref[...], 0.0)
    u = jnp.dot(wn, lw1_ref[...],
                preferred_element_type=jnp.float32) + lbb1_ref[...]
    u_ref[...] = u
    part = jnp.concatenate(
        [jnp.sum(u, axis=0, keepdims=True),
         jnp.sum(u * u, axis=0, keepdims=True)], axis=0)

    @pl.when(i == 0)
    def _():
        us_ref[...] = jnp.zeros_like(us_ref)

    us_ref[...] += part


def _upass(kg, a, h, se, sh, ew2, eb2r, s1, t1, lw1, lbb1r):
    return pl.pallas_call(
        _u_body,
        grid=(_NBLK,),
        in_specs=[
            pl.BlockSpec((_BK, _C), lambda i: (i, 0)),
            pl.BlockSpec((_BN, _C), lambda i: (i, 0)),
            pl.BlockSpec((_BK, 16), lambda i: (i, 0)),
            pl.BlockSpec((1, 16), lambda i: (0, 0)),
            pl.BlockSpec((1, 16), lambda i: (0, 0)),
            pl.BlockSpec((16, _C), lambda i: (0, 0)),
            pl.BlockSpec((1, _C), lambda i: (0, 0)),
            pl.BlockSpec((1, _C), lambda i: (0, 0)),
            pl.BlockSpec((1, _C), lambda i: (0, 0)),
            pl.BlockSpec((_C, 16), lambda i: (0, 0)),
            pl.BlockSpec((1, 16), lambda i: (0, 0)),
        ],
        out_specs=[
            pl.BlockSpec((_BK, 16), lambda i: (i, 0)),
            pl.BlockSpec((2, 16), lambda i: (0, 0)),
        ],
        out_shape=[
            jax.ShapeDtypeStruct((_E, 16), jnp.float32),
            jax.ShapeDtypeStruct((2, 16), jnp.float32),
        ],
    )(kg, a, h, se, sh, ew2, eb2r, s1, t1, lw1, lbb1r)


def _final_body(u_ref, h_ref, vg_ref, vd_ref, dw_ref, s2_ref, t2_ref,
                lw2t_ref, lb2t_ref, se_ref, sh_ref, ew2_ref, eb2_ref, bv_ref,
                o_ref):
    un = jnp.maximum(u_ref[...] * s2_ref[...] + t2_ref[...], 0.0)
    tf = (jnp.dot(un, lw2t_ref[...], preferred_element_type=jnp.float32)
          + lb2t_ref[...]).reshape(_BN, _NS, _C)
    mx = jnp.max(tf, axis=1, keepdims=True)
    ex = jnp.exp(tf - mx)
    watt = ex / jnp.sum(ex, axis=1, keepdims=True)
    hr = jnp.maximum(h_ref[...] * se_ref[...] + sh_ref[...], 0.0)
    emb = jnp.dot(hr, ew2_ref[...],
                  preferred_element_type=jnp.float32) + eb2_ref[...]
    vd = lax.bitcast_convert_type(
        jnp.bitwise_and(vd_ref[...], jnp.int32(-65536)), jnp.float32)
    vg = lax.bitcast_convert_type(
        jnp.bitwise_and(vg_ref[...], jnp.int32(-65536)), jnp.float32)
    vdr = jnp.broadcast_to(vd[:, None, :], (_BN, _NS, _C)).reshape(_BK, _C)
    xv = (vg - vdr + bv_ref[...]) * dw_ref[...]
    z = (xv + emb).reshape(_BN, _NS, _C)
    o_ref[...] = jnp.sum(z * watt, axis=1)


def _final(u, h, vg, v, dwf, s2, t2, lw2t, lb2t, se, sh, ew2, eb2r, bvr):
    return pl.pallas_call(
        _final_body,
        grid=(_NBLK,),
        in_specs=[
            pl.BlockSpec((_BK, 16), lambda i: (i, 0)),
            pl.BlockSpec((_BK, 16), lambda i: (i, 0)),
            pl.BlockSpec((_BK, _C), lambda i: (i, 0)),
            pl.BlockSpec((_BN, _C), lambda i: (i, 0)),
            pl.BlockSpec((_BK, 1), lambda i: (i, 0)),
            pl.BlockSpec((1, 16), lambda i: (0, 0)),
            pl.BlockSpec((1, 16), lambda i: (0, 0)),
            pl.BlockSpec((16, _C), lambda i: (0, 0)),
            pl.BlockSpec((1, _C), lambda i: (0, 0)),
            pl.BlockSpec((1, 16), lambda i: (0, 0)),
            pl.BlockSpec((1, 16), lambda i: (0, 0)),
            pl.BlockSpec((16, _C), lambda i: (0, 0)),
            pl.BlockSpec((1, _C), lambda i: (0, 0)),
            pl.BlockSpec((1, _C), lambda i: (0, 0)),
        ],
        out_specs=pl.BlockSpec((_BN, _C), lambda i: (i, 0)),
        out_shape=jax.ShapeDtypeStruct((_N, _C), jnp.float32),
    )(u, h, vg, v, dwf, s2, t2, lw2t, lb2t, se, sh, ew2, eb2r, bvr)


def _bn_affine(sums, g, b, count):
    mean = sums[0] / count
    var = sums[1] / count - mean * mean
    scale = g / jnp.sqrt(var + 1e-5)
    shift = b - mean * scale
    return scale.reshape(1, -1), shift.reshape(1, -1)


def kernel(p, n, x, idx, Wq, bq, Wk, bk, Wv, bv, eW1, eb1, eg1, ebe1, eW2,
           eb2, lg1, lb1, lW1, lbb1, lg2, lbe2, lW2, lb2):
    idxf = idx.reshape(-1).astype(jnp.int32)
    idx1d = jnp.concatenate([idxf, jnp.zeros((_EP - _E,), jnp.int32)])

    wqk = Wq + Wk
    bqk = (bq - bk).reshape(1, _C)
    a, kv, g = _dense(x, p, n, wqk, Wk, Wv, bqk)

    eye8 = jnp.eye(8, dtype=jnp.float32)
    tile16 = jnp.tile(jnp.eye(16, dtype=jnp.float32), (1, 8))
    blkp = jnp.zeros((16, 16), jnp.float32).at[0:3, :].set(1.0)
    blkn = jnp.zeros((16, 16), jnp.float32).at[5:8, :].set(1.0)
    blkt = jnp.zeros((16, 16), jnp.float32).at[4, :].set(1.0)
    blk23 = (jnp.zeros((16, 16), jnp.float32)
             .at[2, :].set(eW1[2]).at[3, :].set(eW1[3]))
    bp = jnp.kron(eye8, blkp)
    bnm = jnp.kron(eye8, blkn)
    bt = jnp.kron(eye8, blkt)
    b23w = jnp.kron(eye8, blk23)
    cmask = jnp.kron(eye8, jnp.zeros((16, 1), jnp.float32).at[0, 0].set(1.0))
    w1r0t = jnp.tile(eW1[0:1, :], (1, 8))
    w1r1t = jnp.tile(eW1[1:2, :], (1, 8))
    eb1t = jnp.tile(eb1.reshape(1, 16), (1, 8))

    gg = _sc_gather(g, idx1d, 16, tc_tiling=False)
    gg8 = gg.reshape(_EP // 8, _C)
    h8, dwc, hs128 = _edge(gg8, g, tile16, bp, bnm, bt, b23w, w1r0t, w1r1t,
                           eb1t, cmask)
    h = h8.reshape(_E, 16)
    dwf = dwc.reshape(_E, 1)
    hs = hs128.reshape(2, 8, 16).sum(axis=1)
    se, sh = _bn_affine(hs, eg1, ebe1, float(_E))

    kvg = _sc_gather(kv, idx1d, _C, dtype=jnp.int32)
    eb2r = eb2.reshape(1, _C)
    ws = _wstats(kvg, a, h, se, sh, eW2, eb2r)
    s1, t1 = _bn_affine(ws, lg1, lb1, float(_E))

    u, us = _upass(kvg, a, h, se, sh, eW2, eb2r, s1, t1, lW1,
                   lbb1.reshape(1, 16))
    s2, t2 = _bn_affine(us, lg2, lbe2, float(_E))

    lw2t = jnp.dot(lW2, tile16)
    lb2t = jnp.tile(lb2.reshape(1, 16), (1, 8))
    out = _final(u, h, kvg, kv, dwf, s2, t2, lw2t, lb2t, se, sh,
                 eW2, eb2r, bv.reshape(1, _C))
    return out


# speedup vs baseline: 2.2958x; 1.0065x over previous
import functools

import jax
import jax.numpy as jnp
from jax import lax
from jax.experimental import pallas as pl
from jax.experimental.pallas import tpu as pltpu
from jax.experimental.pallas import tpu_sc as plsc

_N = 50000
_NS = 8
_C = 128
_E = _N * _NS
_GB = 128
_NW = 32
_EP = 409600
_ROWS_W = _EP // _NW
_KB = _ROWS_W // _GB
_BN = 1000
_BK = _BN * _NS
_NBLK = _N // _BN

_PI = 3.141592653589793
_TWO_PI = 6.283185307179586


def _sc_gather(table, idx1d, d, tc_tiling=True, dtype=jnp.float32):
    mesh = plsc.VectorSubcoreMesh(core_axis_name="c", subcore_axis_name="s")

    @functools.partial(
        pl.kernel,
        mesh=mesh,
        compiler_params=pltpu.CompilerParams(use_tc_tiling_on_sc=tc_tiling),
        out_type=jax.ShapeDtypeStruct((_EP, d), dtype),
        scratch_types=[
            pltpu.VMEM((_ROWS_W,), jnp.int32),
            pltpu.VMEM((_GB, d), dtype),
            pltpu.VMEM((_GB, d), dtype),
            pltpu.VMEM((_GB, d), dtype),
            pltpu.VMEM((_GB, d), dtype),
            pltpu.SemaphoreType.DMA,
            pltpu.SemaphoreType.DMA,
            pltpu.SemaphoreType.DMA,
            pltpu.SemaphoreType.DMA,
            pltpu.SemaphoreType.DMA,
            pltpu.SemaphoreType.DMA,
            pltpu.SemaphoreType.DMA,
            pltpu.SemaphoreType.DMA,
        ],
    )
    def k(table_hbm, idx_hbm, out_hbm, idx_v, b0, b1, b2, b3,
          g0, g1, g2, g3, o0, o1, o2, o3):
        wid = lax.axis_index("s") * 2 + lax.axis_index("c")
        base = wid * _ROWS_W
        pltpu.sync_copy(idx_hbm.at[pl.ds(base, _ROWS_W)], idx_v)
        bufs = (b0, b1, b2, b3)
        gsems = (g0, g1, g2, g3)
        osems = (o0, o1, o2, o3)
        for j in range(3):
            pltpu.async_copy(table_hbm.at[idx_v.at[pl.ds(j * _GB, _GB)]],
                             bufs[j], gsems[j])

        def body(j4, carry):
            j0 = j4 * 4
            for b in range(4):
                jb = j0 + b
                nb = (b + 3) % 4

                @pl.when(jb + 3 < _KB)
                def _():
                    @pl.when(jb >= 1)
                    def _():
                        pltpu.make_async_copy(
                            bufs[nb], out_hbm.at[pl.ds(0, _GB)],
                            osems[nb]).wait()
                    pltpu.async_copy(
                        table_hbm.at[idx_v.at[pl.ds((jb + 3) * _GB, _GB)]],
                        bufs[nb], gsems[nb])

                pltpu.make_async_copy(
                    table_hbm.at[pl.ds(0, _GB)], bufs[b], gsems[b]).wait()
                pltpu.async_copy(
                    bufs[b], out_hbm.at[pl.ds(base + jb * _GB, _GB)],
                    osems[b])
            return carry

        lax.fori_loop(0, _KB // 4, body, 0)
        for b in range(4):
            pltpu.make_async_copy(
                bufs[b], out_hbm.at[pl.ds(0, _GB)], osems[b]).wait()

    return k(table, idx1d)


def _bf16_bits(f):
    b = lax.bitcast_convert_type(f, jnp.int32)
    return b + 0x7FFF + jnp.bitwise_and(jnp.right_shift(b, 16), 1)


def _dense_body(x_ref, p_ref, n_ref, wqk_ref, wk_ref, wv_ref, bqk_ref,
                a_ref, kv_ref, g_ref):
    xb = x_ref[...]
    a_ref[...] = jnp.dot(xb, wqk_ref[...],
                         preferred_element_type=jnp.float32) + bqk_ref[...]
    kb = _bf16_bits(jnp.dot(xb, wk_ref[...],
                            preferred_element_type=jnp.float32))
    vb = _bf16_bits(jnp.dot(xb, wv_ref[...],
                            preferred_element_type=jnp.float32))
    klo = jnp.bitwise_and(jnp.right_shift(kb, 16), 0xFFFF)
    vhi = jnp.bitwise_and(vb, jnp.int32(-65536))
    kv_ref[...] = jnp.bitwise_or(klo, vhi)
    pb = p_ref[...]
    nb = n_ref[...]
    px = pb[:, 0:1]
    py = pb[:, 1:2]
    r = jnp.sqrt(px * px + py * py + 1e-12)
    theta = jnp.arctan2(py, px)
    zpad = jnp.zeros((_BN, 8), jnp.float32)
    g_ref[...] = jnp.concatenate([pb, r, theta, nb, zpad], axis=1)


def _dense(x, p, n, wqk, wk, wv, bqk):
    return pl.pallas_call(
        _dense_body,
        grid=(_NBLK,),
        in_specs=[
            pl.BlockSpec((_BN, _C), lambda i: (i, 0)),
            pl.BlockSpec((_BN, 3), lambda i: (i, 0)),
            pl.BlockSpec((_BN, 3), lambda i: (i, 0)),
            pl.BlockSpec((_C, _C), lambda i: (0, 0)),
            pl.BlockSpec((_C, _C), lambda i: (0, 0)),
            pl.BlockSpec((_C, _C), lambda i: (0, 0)),
            pl.BlockSpec((1, _C), lambda i: (0, 0)),
        ],
        out_specs=[
            pl.BlockSpec((_BN, _C), lambda i: (i, 0)),
            pl.BlockSpec((_BN, _C), lambda i: (i, 0)),
            pl.BlockSpec((_BN, 16), lambda i: (i, 0)),
        ],
        out_shape=[
            jax.ShapeDtypeStruct((_N, _C), jnp.float32),
            jax.ShapeDtypeStruct((_N, _C), jnp.int32),
            jax.ShapeDtypeStruct((_N, 16), jnp.float32),
        ],
    )(x, p, n, wqk, wk, wv, bqk)


def _edge_body(gg8_ref, gd_ref, tile16_ref, bp_ref, bn_ref, bt_ref, b23w_ref,
               w1r0t_ref, w1r1t_ref, eb1t_ref, cmask_ref,
               h8_ref, dwc_ref, hs_ref):
    i = pl.program_id(0)
    gg8 = gg8_ref[...]
    gdr = jnp.dot(gd_ref[...], tile16_ref[...],
                  preferred_element_type=jnp.float32)
    dd = gg8 - gdr
    s = dd * dd
    t = jnp.dot(s, bp_ref[...], preferred_element_type=jnp.float32) + 1e-12
    dist = jnp.sqrt(t)
    dwfull = jnp.exp(-dist)
    dwc_ref[...] = jnp.dot(dwfull, cmask_ref[...],
                           preferred_element_type=jnp.float32)
    sn = jnp.dot(s, bn_ref[...], preferred_element_type=jnp.float32) + 1e-12
    dn = jnp.sqrt(sn)
    th = jnp.dot(dd, bt_ref[...], preferred_element_type=jnp.float32)
    y = th + _PI
    mm = y - _TWO_PI * jnp.floor(y * (1.0 / _TWO_PI))
    dtheta = jnp.abs(mm - _PI)
    hzr = jnp.dot(jnp.abs(dd), b23w_ref[...],
                  preferred_element_type=jnp.float32)
    h = dn * w1r0t_ref[...] + dtheta * w1r1t_ref[...] + hzr + eb1t_ref[...]
    h8_ref[...] = h
    part = jnp.concatenate(
        [jnp.sum(h, axis=0, keepdims=True),
         jnp.sum(h * h, axis=0, keepdims=True)], axis=0)

    @pl.when(i == 0)
    def _():
        hs_ref[...] = jnp.zeros_like(hs_ref)

    hs_ref[...] += part


def _edge(gg8, g, tile16, bp, bnm, bt, b23w, w1r0t, w1r1t, eb1t, cmask):
    return pl.pallas_call(
        _edge_body,
        grid=(_NBLK,),
        in_specs=[
            pl.BlockSpec((_BN, _C), lambda i: (i, 0)),
            pl.BlockSpec((_BN, 16), lambda i: (i, 0)),
            pl.BlockSpec((16, _C), lambda i: (0, 0)),
            pl.BlockSpec((_C, _C), lambda i: (0, 0)),
            pl.BlockSpec((_C, _C), lambda i: (0, 0)),
            pl.BlockSpec((_C, _C), lambda i: (0, 0)),
            pl.BlockSpec((_C, _C), lambda i: (0, 0)),
            pl.BlockSpec((1, _C), lambda i: (0, 0)),
            pl.BlockSpec((1, _C), lambda i: (0, 0)),
            pl.BlockSpec((1, _C), lambda i: (0, 0)),
            pl.BlockSpec((_C, 8), lambda i: (0, 0)),
        ],
        out_specs=[
            pl.BlockSpec((_BN, _C), lambda i: (i, 0)),
            pl.BlockSpec((_BN, 8), lambda i: (i, 0)),
            pl.BlockSpec((2, _C), lambda i: (0, 0)),
        ],
        out_shape=[
            jax.ShapeDtypeStruct((_E // 8, _C), jnp.float32),
            jax.ShapeDtypeStruct((_N, 8), jnp.float32),
            jax.ShapeDtypeStruct((2, _C), jnp.float32),
        ],
    )(gg8, g, tile16, bp, bnm, bt, b23w, w1r0t, w1r1t, eb1t, cmask)


def _wstats_body(kg_ref, a_ref, h_ref, se_ref, sh_ref, ew2_ref, eb2_ref,
                 ws_ref):
    i = pl.program_id(0)
    hr = jnp.maximum(h_ref[...] * se_ref[...] + sh_ref[...], 0.0)
    emb = jnp.dot(hr, ew2_ref[...],
                  preferred_element_type=jnp.float32) + eb2_ref[...]
    a = a_ref[...]
    ar = jnp.broadcast_to(a[:, None, :], (_BN, _NS, _C)).reshape(_BK, _C)
    kg = lax.bitcast_convert_type(
        jnp.left_shift(kg_ref[...], 16), jnp.float32)
    w = ar - kg + emb
    part = jnp.concatenate(
        [jnp.sum(w, axis=0, keepdims=True),
         jnp.sum(w * w, axis=0, keepdims=True)], axis=0)

    @pl.when(i == 0)
    def _():
        ws_ref[...] = jnp.zeros_like(ws_ref)

    ws_ref[...] += part


def _wstats(kg, a, h, se, sh, ew2, eb2r):
    return pl.pallas_call(
        _wstats_body,
        grid=(_NBLK,),
        in_specs=[
            pl.BlockSpec((_BK, _C), lambda i: (i, 0)),
            pl.BlockSpec((_BN, _C), lambda i: (i, 0)),
            pl.BlockSpec((_BK, 16), lambda i: (i, 0)),
            pl.BlockSpec((1, 16), lambda i: (0, 0)),
            pl.BlockSpec((1, 16), lambda i: (0, 0)),
            pl.BlockSpec((16, _C), lambda i: (0, 0)),
            pl.BlockSpec((1, _C), lambda i: (0, 0)),
        ],
        out_specs=pl.BlockSpec((2, _C), lambda i: (0, 0)),
        out_shape=jax.ShapeDtypeStruct((2, _C), jnp.float32),
    )(kg, a, h, se, sh, ew2, eb2r)


def _u_body(kg_ref, a_ref, h_ref, se_ref, sh_ref, ew2_ref, eb2_ref,
            s1_ref, t1_ref, lw1_ref, lbb1_ref, u_ref, us_ref):
    i = pl.program_id(0)
    hr = jnp.maximum(h_ref[...] * se_ref[...] + sh_ref[...], 0.0)
    emb = jnp.dot(hr, ew2_ref[...],
                  preferred_element_type=jnp.float32) + eb2_ref[...]
    a = a_ref[...]
    ar = jnp.broadcast_to(a[:, None, :], (_BN, _NS, _C)).reshape(_BK, _C)
    kg = lax.bitcast_convert_type(
        jnp.left_shift(kg_ref[...], 16), jnp.float32)
    w = ar - kg + emb
    wn = jnp.maximum(w * s1_ref[...] + t1_ref[...], 0.0)
    u = jnp.dot(wn, lw1_ref[...],
                preferred_element_type=jnp.float32) + lbb1_ref[...]
    u_ref[...] = u
    part = jnp.concatenate(
        [jnp.sum(u, axis=0, keepdims=True),
         jnp.sum(u * u, axis=0, keepdims=True)], axis=0)

    @pl.when(i == 0)
    def _():
        us_ref[...] = jnp.zeros_like(us_ref)

    us_ref[...] += part


def _upass(kg, a, h, se, sh, ew2, eb2r, s1, t1, lw1, lbb1r):
    return pl.pallas_call(
        _u_body,
        grid=(_NBLK,),
        in_specs=[
            pl.BlockSpec((_BK, _C), lambda i: (i, 0)),
            pl.BlockSpec((_BN, _C), lambda i: (i, 0)),
            pl.BlockSpec((_BK, 16), lambda i: (i, 0)),
            pl.BlockSpec((1, 16), lambda i: (0, 0)),
            pl.BlockSpec((1, 16), lambda i: (0, 0)),
            pl.BlockSpec((16, _C), lambda i: (0, 0)),
            pl.BlockSpec((1, _C), lambda i: (0, 0)),
            pl.BlockSpec((1, _C), lambda i: (0, 0)),
            pl.BlockSpec((1, _C), lambda i: (0, 0)),
            pl.BlockSpec((_C, 16), lambda i: (0, 0)),
            pl.BlockSpec((1, 16), lambda i: (0, 0)),
        ],
        out_specs=[
            pl.BlockSpec((_BK, 16), lambda i: (i, 0)),
            pl.BlockSpec((2, 16), lambda i: (0, 0)),
        ],
        out_shape=[
            jax.ShapeDtypeStruct((_E, 16), jnp.float32),
            jax.ShapeDtypeStruct((2, 16), jnp.float32),
        ],
    )(kg, a, h, se, sh, ew2, eb2r, s1, t1, lw1, lbb1r)


def _final_body(u_ref, h_ref, vg_ref, vd_ref, dw_ref, s2_ref, t2_ref,
                lw2t_ref, lb2t_ref, se_ref, sh_ref, ew2_ref, eb2_ref, bv_ref,
                o_ref):
    un = jnp.maximum(u_ref[...] * s2_ref[...] + t2_ref[...], 0.0)
    tf = (jnp.dot(un, lw2t_ref[...], preferred_element_type=jnp.float32)
          + lb2t_ref[...]).reshape(_BN, _NS, _C)
    mx = jnp.max(tf, axis=1, keepdims=True)
    ex = jnp.exp(tf - mx)
    watt = ex / jnp.sum(ex, axis=1, keepdims=True)
    hr = jnp.maximum(h_ref[...] * se_ref[...] + sh_ref[...], 0.0)
    emb = jnp.dot(hr, ew2_ref[...],
                  preferred_element_type=jnp.float32) + eb2_ref[...]
    vd = lax.bitcast_convert_type(
        jnp.bitwise_and(vd_ref[...], jnp.int32(-65536)), jnp.float32)
    vg = lax.bitcast_convert_type(
        jnp.bitwise_and(vg_ref[...], jnp.int32(-65536)), jnp.float32)
    vdr = jnp.broadcast_to(vd[:, None, :], (_BN, _NS, _C)).reshape(_BK, _C)
    xv = (vg - vdr + bv_ref[...]) * dw_ref[...]
    z = (xv + emb).reshape(_BN, _NS, _C)
    o_ref[...] = jnp.sum(z * watt, axis=1)


def _final(u, h, vg, v, dwf, s2, t2, lw2t, lb2t, se, sh, ew2, eb2r, bvr):
    return pl.pallas_call(
        _final_body,
        grid=(_NBLK,),
        in_specs=[
            pl.BlockSpec((_BK, 16), lambda i: (i, 0)),
            pl.BlockSpec((_BK, 16), lambda i: (i, 0)),
            pl.BlockSpec((_BK, _C), lambda i: (i, 0)),
            pl.BlockSpec((_BN, _C), lambda i: (i, 0)),
            pl.BlockSpec((_BK, 1), lambda i: (i, 0)),
            pl.BlockSpec((1, 16), lambda i: (0, 0)),
            pl.BlockSpec((1, 16), lambda i: (0, 0)),
            pl.BlockSpec((16, _C), lambda i: (0, 0)),
            pl.BlockSpec((1, _C), lambda i: (0, 0)),
            pl.BlockSpec((1, 16), lambda i: (0, 0)),
            pl.BlockSpec((1, 16), lambda i: (0, 0)),
            pl.BlockSpec((16, _C), lambda i: (0, 0)),
            pl.BlockSpec((1, _C), lambda i: (0, 0)),
            pl.BlockSpec((1, _C), lambda i: (0, 0)),
        ],
        out_specs=pl.BlockSpec((_BN, _C), lambda i: (i, 0)),
        out_shape=jax.ShapeDtypeStruct((_N, _C), jnp.float32),
    )(u, h, vg, v, dwf, s2, t2, lw2t, lb2t, se, sh, ew2, eb2r, bvr)


def _bn_affine(sums, g, b, count):
    mean = sums[0] / count
    var = sums[1] / count - mean * mean
    scale = g / jnp.sqrt(var + 1e-5)
    shift = b - mean * scale
    return scale.reshape(1, -1), shift.reshape(1, -1)


def kernel(p, n, x, idx, Wq, bq, Wk, bk, Wv, bv, eW1, eb1, eg1, ebe1, eW2,
           eb2, lg1, lb1, lW1, lbb1, lg2, lbe2, lW2, lb2):
    idxf = idx.reshape(-1).astype(jnp.int32)
    idx1d = jnp.concatenate([idxf, jnp.zeros((_EP - _E,), jnp.int32)])

    wqk = Wq + Wk
    bqk = (bq - bk).reshape(1, _C)
    a, kv, g = _dense(x, p, n, wqk, Wk, Wv, bqk)

    eye8 = jnp.eye(8, dtype=jnp.float32)
    tile16 = jnp.tile(jnp.eye(16, dtype=jnp.float32), (1, 8))
    blkp = jnp.zeros((16, 16), jnp.float32).at[0:3, :].set(1.0)
    blkn = jnp.zeros((16, 16), jnp.float32).at[5:8, :].set(1.0)
    blkt = jnp.zeros((16, 16), jnp.float32).at[4, :].set(1.0)
    blk23 = (jnp.zeros((16, 16), jnp.float32)
             .at[2, :].set(eW1[2]).at[3, :].set(eW1[3]))
    bp = jnp.kron(eye8, blkp)
    bnm = jnp.kron(eye8, blkn)
    bt = jnp.kron(eye8, blkt)
    b23w = jnp.kron(eye8, blk23)
    cmask = jnp.kron(eye8, jnp.zeros((16, 1), jnp.float32).at[0, 0].set(1.0))
    w1r0t = jnp.tile(eW1[0:1, :], (1, 8))
    w1r1t = jnp.tile(eW1[1:2, :], (1, 8))
    eb1t = jnp.tile(eb1.reshape(1, 16), (1, 8))

    gg = _sc_gather(g, idx1d, 16, tc_tiling=False)
    gg8 = gg.reshape(_EP // 8, _C)
    h8, dwc, hs128 = _edge(gg8, g, tile16, bp, bnm, bt, b23w, w1r0t, w1r1t,
                           eb1t, cmask)
    h = h8.reshape(_E, 16)
    dwf = dwc.reshape(_E, 1)
    hs = hs128.reshape(2, 8, 16).sum(axis=1)
    se, sh = _bn_affine(hs, eg1, ebe1, float(_E))

    kvg = _sc_gather(kv, idx1d, _C, dtype=jnp.int32)
    eb2r = eb2.reshape(1, _C)
    ws = _wstats(kvg, a, h, se, sh, eW2, eb2r)
    s1, t1 = _bn_affine(ws, lg1, lb1, float(_E))

    u, us = _upass(kvg, a, h, se, sh, eW2, eb2r, s1, t1, lW1,
                   lbb1.reshape(1, 16))
    s2, t2 = _bn_affine(us, lg2, lbe2, float(_E))

    lw2t = jnp.dot(lW2, tile16)
    lb2t = jnp.tile(lb2.reshape(1, 16), (1, 8))
    out = _final(u, h, kvg, kv, dwf, s2, t2, lw2t, lb2t, se, sh,
                 eW2, eb2r, bv.reshape(1, _C))
    return out


# speedup vs baseline: 2.9780x; 1.2971x over previous
import functools

import jax
import jax.numpy as jnp
from jax import lax
from jax.experimental import pallas as pl
from jax.experimental.pallas import tpu as pltpu
from jax.experimental.pallas import tpu_sc as plsc

_N = 50000
_NS = 8
_C = 128
_E = _N * _NS
_GB = 128
_NW = 32
_EP = 409600
_ROWS_W = _EP // _NW
_KB = _ROWS_W // _GB
_BN = 1000
_BK = _BN * _NS
_NBLK = _N // _BN

_PI = 3.141592653589793
_TWO_PI = 6.283185307179586


def _sc_gather(table, idx1d, d, tc_tiling=True, dtype=jnp.float32):
    mesh = plsc.VectorSubcoreMesh(core_axis_name="c", subcore_axis_name="s")

    @functools.partial(
        pl.kernel,
        mesh=mesh,
        compiler_params=pltpu.CompilerParams(use_tc_tiling_on_sc=tc_tiling),
        out_type=jax.ShapeDtypeStruct((_EP, d), dtype),
        scratch_types=[
            pltpu.VMEM((_ROWS_W,), jnp.int32),
            pltpu.VMEM((_GB, d), dtype),
            pltpu.VMEM((_GB, d), dtype),
            pltpu.VMEM((_GB, d), dtype),
            pltpu.VMEM((_GB, d), dtype),
            pltpu.SemaphoreType.DMA,
            pltpu.SemaphoreType.DMA,
            pltpu.SemaphoreType.DMA,
            pltpu.SemaphoreType.DMA,
            pltpu.SemaphoreType.DMA,
            pltpu.SemaphoreType.DMA,
            pltpu.SemaphoreType.DMA,
            pltpu.SemaphoreType.DMA,
        ],
    )
    def k(table_hbm, idx_hbm, out_hbm, idx_v, b0, b1, b2, b3,
          g0, g1, g2, g3, o0, o1, o2, o3):
        wid = lax.axis_index("s") * 2 + lax.axis_index("c")
        base = wid * _ROWS_W
        pltpu.sync_copy(idx_hbm.at[pl.ds(base, _ROWS_W)], idx_v)
        bufs = (b0, b1, b2, b3)
        gsems = (g0, g1, g2, g3)
        osems = (o0, o1, o2, o3)
        for j in range(3):
            pltpu.async_copy(table_hbm.at[idx_v.at[pl.ds(j * _GB, _GB)]],
                             bufs[j], gsems[j])

        def body(j4, carry):
            j0 = j4 * 4
            for b in range(4):
                jb = j0 + b
                nb = (b + 3) % 4

                @pl.when(jb + 3 < _KB)
                def _():
                    @pl.when(jb >= 1)
                    def _():
                        pltpu.make_async_copy(
                            bufs[nb], out_hbm.at[pl.ds(0, _GB)],
                            osems[nb]).wait()
                    pltpu.async_copy(
                        table_hbm.at[idx_v.at[pl.ds((jb + 3) * _GB, _GB)]],
                        bufs[nb], gsems[nb])

                pltpu.make_async_copy(
                    table_hbm.at[pl.ds(0, _GB)], bufs[b], gsems[b]).wait()
                pltpu.async_copy(
                    bufs[b], out_hbm.at[pl.ds(base + jb * _GB, _GB)],
                    osems[b])
            return carry

        lax.fori_loop(0, _KB // 4, body, 0)
        for b in range(4):
            pltpu.make_async_copy(
                bufs[b], out_hbm.at[pl.ds(0, _GB)], osems[b]).wait()

    return k(table, idx1d)


def _bf16_bits(f):
    b = lax.bitcast_convert_type(f, jnp.int32)
    return b + 0x7FFF + jnp.bitwise_and(jnp.right_shift(b, 16), 1)


def _dense_body(x_ref, p_ref, n_ref, wqk_ref, wk_ref, wv_ref, bqk_ref,
                a_ref, kv_ref, g_ref):
    xb = x_ref[...]
    a_ref[...] = jnp.dot(xb, wqk_ref[...],
                         preferred_element_type=jnp.float32) + bqk_ref[...]
    kb = _bf16_bits(jnp.dot(xb, wk_ref[...],
                            preferred_element_type=jnp.float32))
    vb = _bf16_bits(jnp.dot(xb, wv_ref[...],
                            preferred_element_type=jnp.float32))
    klo = jnp.bitwise_and(jnp.right_shift(kb, 16), 0xFFFF)
    vhi = jnp.bitwise_and(vb, jnp.int32(-65536))
    kv_ref[...] = jnp.bitwise_or(klo, vhi)
    pb = p_ref[...]
    nb = n_ref[...]
    px = pb[:, 0:1]
    py = pb[:, 1:2]
    r = jnp.sqrt(px * px + py * py + 1e-12)
    theta = jnp.arctan2(py, px)
    zpad = jnp.zeros((_BN, 8), jnp.float32)
    g_ref[...] = jnp.concatenate([pb, r, theta, nb, zpad], axis=1)


def _dense(x, p, n, wqk, wk, wv, bqk):
    return pl.pallas_call(
        _dense_body,
        grid=(_NBLK,),
        in_specs=[
            pl.BlockSpec((_BN, _C), lambda i: (i, 0)),
            pl.BlockSpec((_BN, 3), lambda i: (i, 0)),
            pl.BlockSpec((_BN, 3), lambda i: (i, 0)),
            pl.BlockSpec((_C, _C), lambda i: (0, 0)),
            pl.BlockSpec((_C, _C), lambda i: (0, 0)),
            pl.BlockSpec((_C, _C), lambda i: (0, 0)),
            pl.BlockSpec((1, _C), lambda i: (0, 0)),
        ],
        out_specs=[
            pl.BlockSpec((_BN, _C), lambda i: (i, 0)),
            pl.BlockSpec((_BN, _C), lambda i: (i, 0)),
            pl.BlockSpec((_BN, 16), lambda i: (i, 0)),
        ],
        out_shape=[
            jax.ShapeDtypeStruct((_N, _C), jnp.float32),
            jax.ShapeDtypeStruct((_N, _C), jnp.int32),
            jax.ShapeDtypeStruct((_N, 16), jnp.float32),
        ],
    )(x, p, n, wqk, wk, wv, bqk)


def _edge_body(gg8_ref, gd_ref, tile16_ref, bp_ref, bn_ref, bt_ref, b23w_ref,
               w1r0t_ref, w1r1t_ref, eb1t_ref, cmask_ref,
               h8_ref, dwc_ref, hs_ref):
    i = pl.program_id(0)
    gg8 = gg8_ref[...]
    gdr = jnp.dot(gd_ref[...], tile16_ref[...],
                  preferred_element_type=jnp.float32)
    dd = gg8 - gdr
    s = dd * dd
    t = jnp.dot(s, bp_ref[...], preferred_element_type=jnp.float32) + 1e-12
    dist = jnp.sqrt(t)
    dwfull = jnp.exp(-dist)
    dwc_ref[...] = jnp.dot(dwfull, cmask_ref[...],
                           preferred_element_type=jnp.float32)
    sn = jnp.dot(s, bn_ref[...], preferred_element_type=jnp.float32) + 1e-12
    dn = jnp.sqrt(sn)
    th = jnp.dot(dd, bt_ref[...], preferred_element_type=jnp.float32)
    y = th + _PI
    mm = y - _TWO_PI * jnp.floor(y * (1.0 / _TWO_PI))
    dtheta = jnp.abs(mm - _PI)
    hzr = jnp.dot(jnp.abs(dd), b23w_ref[...],
                  preferred_element_type=jnp.float32)
    h = dn * w1r0t_ref[...] + dtheta * w1r1t_ref[...] + hzr + eb1t_ref[...]
    h8_ref[...] = h
    part = jnp.concatenate(
        [jnp.sum(h, axis=0, keepdims=True),
         jnp.sum(h * h, axis=0, keepdims=True)], axis=0)

    @pl.when(i == 0)
    def _():
        hs_ref[...] = jnp.zeros_like(hs_ref)

    hs_ref[...] += part


def _edge(gg8, g, tile16, bp, bnm, bt, b23w, w1r0t, w1r1t, eb1t, cmask):
    return pl.pallas_call(
        _edge_body,
        grid=(_NBLK,),
        in_specs=[
            pl.BlockSpec((_BN, _C), lambda i: (i, 0)),
            pl.BlockSpec((_BN, 16), lambda i: (i, 0)),
            pl.BlockSpec((16, _C), lambda i: (0, 0)),
            pl.BlockSpec((_C, _C), lambda i: (0, 0)),
            pl.BlockSpec((_C, _C), lambda i: (0, 0)),
            pl.BlockSpec((_C, _C), lambda i: (0, 0)),
            pl.BlockSpec((_C, _C), lambda i: (0, 0)),
            pl.BlockSpec((1, _C), lambda i: (0, 0)),
            pl.BlockSpec((1, _C), lambda i: (0, 0)),
            pl.BlockSpec((1, _C), lambda i: (0, 0)),
            pl.BlockSpec((_C, 8), lambda i: (0, 0)),
        ],
        out_specs=[
            pl.BlockSpec((_BN, _C), lambda i: (i, 0)),
            pl.BlockSpec((_BN, 8), lambda i: (i, 0)),
            pl.BlockSpec((2, _C), lambda i: (0, 0)),
        ],
        out_shape=[
            jax.ShapeDtypeStruct((_E // 8, _C), jnp.float32),
            jax.ShapeDtypeStruct((_N, 8), jnp.float32),
            jax.ShapeDtypeStruct((2, _C), jnp.float32),
        ],
    )(gg8, g, tile16, bp, bnm, bt, b23w, w1r0t, w1r1t, eb1t, cmask)


def _wstats_body(kg_ref, a_ref, h_ref, se_ref, sh_ref, ew2_ref, eb2_ref,
                 ws_ref):
    i = pl.program_id(0)
    hr = jnp.maximum(h_ref[...] * se_ref[...] + sh_ref[...], 0.0)
    emb = jnp.dot(hr, ew2_ref[...],
                  preferred_element_type=jnp.float32) + eb2_ref[...]
    a = a_ref[...]
    ar = jnp.broadcast_to(a[:, None, :], (_BN, _NS, _C)).reshape(_BK, _C)
    kg = lax.bitcast_convert_type(
        jnp.left_shift(kg_ref[...], 16), jnp.float32)
    w = ar - kg + emb
    part = jnp.concatenate(
        [jnp.sum(w, axis=0, keepdims=True),
         jnp.sum(w * w, axis=0, keepdims=True)], axis=0)

    @pl.when(i == 0)
    def _():
        ws_ref[...] = jnp.zeros_like(ws_ref)

    ws_ref[...] += part


def _wstats(kg, a, h, se, sh, ew2, eb2r):
    return pl.pallas_call(
        _wstats_body,
        grid=(_NBLK,),
        in_specs=[
            pl.BlockSpec((_BK, _C), lambda i: (i, 0)),
            pl.BlockSpec((_BN, _C), lambda i: (i, 0)),
            pl.BlockSpec((_BK, 16), lambda i: (i, 0)),
            pl.BlockSpec((1, 16), lambda i: (0, 0)),
            pl.BlockSpec((1, 16), lambda i: (0, 0)),
            pl.BlockSpec((16, _C), lambda i: (0, 0)),
            pl.BlockSpec((1, _C), lambda i: (0, 0)),
        ],
        out_specs=pl.BlockSpec((2, _C), lambda i: (0, 0)),
        out_shape=jax.ShapeDtypeStruct((2, _C), jnp.float32),
    )(kg, a, h, se, sh, ew2, eb2r)


def _u_body(kg_ref, a_ref, h_ref, se_ref, sh_ref, ew2_ref, eb2_ref,
            s1_ref, t1_ref, lw1_ref, lbb1_ref, u_ref, us_ref):
    i = pl.program_id(0)
    hr = jnp.maximum(h_ref[...] * se_ref[...] + sh_ref[...], 0.0)
    emb = jnp.dot(hr, ew2_ref[...],
                  preferred_element_type=jnp.float32) + eb2_ref[...]
    a = a_ref[...]
    ar = jnp.broadcast_to(a[:, None, :], (_BN, _NS, _C)).reshape(_BK, _C)
    kg = lax.bitcast_convert_type(
        jnp.left_shift(kg_ref[...], 16), jnp.float32)
    w = ar - kg + emb
    wn = jnp.maximum(w * s1_ref[...] + t1_ref[...], 0.0)
    u = jnp.dot(wn, lw1_ref[...],
                preferred_element_type=jnp.float32) + lbb1_ref[...]
    u_ref[...] = u
    part = jnp.concatenate(
        [jnp.sum(u, axis=0, keepdims=True),
         jnp.sum(u * u, axis=0, keepdims=True)], axis=0)

    @pl.when(i == 0)
    def _():
        us_ref[...] = jnp.zeros_like(us_ref)

    us_ref[...] += part


def _upass(kg, a, h, se, sh, ew2, eb2r, s1, t1, lw1, lbb1r):
    return pl.pallas_call(
        _u_body,
        grid=(_NBLK,),
        in_specs=[
            pl.BlockSpec((_BK, _C), lambda i: (i, 0)),
            pl.BlockSpec((_BN, _C), lambda i: (i, 0)),
            pl.BlockSpec((_BK, 16), lambda i: (i, 0)),
            pl.BlockSpec((1, 16), lambda i: (0, 0)),
            pl.BlockSpec((1, 16), lambda i: (0, 0)),
            pl.BlockSpec((16, _C), lambda i: (0, 0)),
            pl.BlockSpec((1, _C), lambda i: (0, 0)),
            pl.BlockSpec((1, _C), lambda i: (0, 0)),
            pl.BlockSpec((1, _C), lambda i: (0, 0)),
            pl.BlockSpec((_C, 16), lambda i: (0, 0)),
            pl.BlockSpec((1, 16), lambda i: (0, 0)),
        ],
        out_specs=[
            pl.BlockSpec((_BK, 16), lambda i: (i, 0)),
            pl.BlockSpec((2, 16), lambda i: (0, 0)),
        ],
        out_shape=[
            jax.ShapeDtypeStruct((_E, 16), jnp.float32),
            jax.ShapeDtypeStruct((2, 16), jnp.float32),
        ],
    )(kg, a, h, se, sh, ew2, eb2r, s1, t1, lw1, lbb1r)


def _final_body(u_ref, h_ref, vg_ref, vd_ref, dw_ref, s2_ref, t2_ref,
                lw2t_ref, lb2t_ref, se_ref, sh_ref, ew2_ref, eb2_ref, bv_ref,
                o_ref):
    un = jnp.maximum(u_ref[...] * s2_ref[...] + t2_ref[...], 0.0)
    tf = (jnp.dot(un, lw2t_ref[...], preferred_element_type=jnp.float32)
          + lb2t_ref[...]).reshape(_BN, _NS, _C)
    mx = jnp.max(tf, axis=1, keepdims=True)
    ex = jnp.exp(tf - mx)
    watt = ex / jnp.sum(ex, axis=1, keepdims=True)
    hr = jnp.maximum(h_ref[...] * se_ref[...] + sh_ref[...], 0.0)
    emb = jnp.dot(hr, ew2_ref[...],
                  preferred_element_type=jnp.float32) + eb2_ref[...]
    vd = lax.bitcast_convert_type(
        jnp.bitwise_and(vd_ref[...], jnp.int32(-65536)), jnp.float32)
    vg = lax.bitcast_convert_type(
        jnp.bitwise_and(vg_ref[...], jnp.int32(-65536)), jnp.float32)
    vdr = jnp.broadcast_to(vd[:, None, :], (_BN, _NS, _C)).reshape(_BK, _C)
    xv = (vg - vdr + bv_ref[...]) * dw_ref[...]
    z = (xv + emb).reshape(_BN, _NS, _C)
    o_ref[...] = jnp.sum(z * watt, axis=1)


def _final(u, h, vg, v, dwf, s2, t2, lw2t, lb2t, se, sh, ew2, eb2r, bvr):
    return pl.pallas_call(
        _final_body,
        grid=(_NBLK,),
        in_specs=[
            pl.BlockSpec((_BK, 16), lambda i: (i, 0)),
            pl.BlockSpec((_BK, 16), lambda i: (i, 0)),
            pl.BlockSpec((_BK, _C), lambda i: (i, 0)),
            pl.BlockSpec((_BN, _C), lambda i: (i, 0)),
            pl.BlockSpec((_BK, 1), lambda i: (i, 0)),
            pl.BlockSpec((1, 16), lambda i: (0, 0)),
            pl.BlockSpec((1, 16), lambda i: (0, 0)),
            pl.BlockSpec((16, _C), lambda i: (0, 0)),
            pl.BlockSpec((1, _C), lambda i: (0, 0)),
            pl.BlockSpec((1, 16), lambda i: (0, 0)),
            pl.BlockSpec((1, 16), lambda i: (0, 0)),
            pl.BlockSpec((16, _C), lambda i: (0, 0)),
            pl.BlockSpec((1, _C), lambda i: (0, 0)),
            pl.BlockSpec((1, _C), lambda i: (0, 0)),
        ],
        out_specs=pl.BlockSpec((_BN, _C), lambda i: (i, 0)),
        out_shape=jax.ShapeDtypeStruct((_N, _C), jnp.float32),
    )(u, h, vg, v, dwf, s2, t2, lw2t, lb2t, se, sh, ew2, eb2r, bvr)


def _bn_affine(sums, g, b, count):
    mean = sums[0] / count
    var = sums[1] / count - mean * mean
    scale = g / jnp.sqrt(var + 1e-5)
    shift = b - mean * scale
    return scale.reshape(1, -1), shift.reshape(1, -1)


def kernel(p, n, x, idx, Wq, bq, Wk, bk, Wv, bv, eW1, eb1, eg1, ebe1, eW2,
           eb2, lg1, lb1, lW1, lbb1, lg2, lbe2, lW2, lb2):
    idxf = idx.reshape(-1).astype(jnp.int32)
    idx1d = jnp.concatenate(
        [idxf, jnp.arange(_EP - _E, dtype=jnp.int32)])

    wqk = Wq + Wk
    bqk = (bq - bk).reshape(1, _C)
    a, kv, g = _dense(x, p, n, wqk, Wk, Wv, bqk)

    eye8 = jnp.eye(8, dtype=jnp.float32)
    tile16 = jnp.tile(jnp.eye(16, dtype=jnp.float32), (1, 8))
    blkp = jnp.zeros((16, 16), jnp.float32).at[0:3, :].set(1.0)
    blkn = jnp.zeros((16, 16), jnp.float32).at[5:8, :].set(1.0)
    blkt = jnp.zeros((16, 16), jnp.float32).at[4, :].set(1.0)
    blk23 = (jnp.zeros((16, 16), jnp.float32)
             .at[2, :].set(eW1[2]).at[3, :].set(eW1[3]))
    bp = jnp.kron(eye8, blkp)
    bnm = jnp.kron(eye8, blkn)
    bt = jnp.kron(eye8, blkt)
    b23w = jnp.kron(eye8, blk23)
    cmask = jnp.kron(eye8, jnp.zeros((16, 1), jnp.float32).at[0, 0].set(1.0))
    w1r0t = jnp.tile(eW1[0:1, :], (1, 8))
    w1r1t = jnp.tile(eW1[1:2, :], (1, 8))
    eb1t = jnp.tile(eb1.reshape(1, 16), (1, 8))

    gg = _sc_gather(g, idx1d, 16, tc_tiling=False)
    gg8 = gg.reshape(_EP // 8, _C)
    h8, dwc, hs128 = _edge(gg8, g, tile16, bp, bnm, bt, b23w, w1r0t, w1r1t,
                           eb1t, cmask)
    h = h8.reshape(_E, 16)
    dwf = dwc.reshape(_E, 1)
    hs = hs128.reshape(2, 8, 16).sum(axis=1)
    se, sh = _bn_affine(hs, eg1, ebe1, float(_E))

    kvg = _sc_gather(kv, idx1d, _C, dtype=jnp.int32)
    eb2r = eb2.reshape(1, _C)
    ws = _wstats(kvg, a, h, se, sh, eW2, eb2r)
    s1, t1 = _bn_affine(ws, lg1, lb1, float(_E))

    u, us = _upass(kvg, a, h, se, sh, eW2, eb2r, s1, t1, lW1,
                   lbb1.reshape(1, 16))
    s2, t2 = _bn_affine(us, lg2, lbe2, float(_E))

    lw2t = jnp.dot(lW2, tile16)
    lb2t = jnp.tile(lb2.reshape(1, 16), (1, 8))
    out = _final(u, h, kvg, kv, dwf, s2, t2, lw2t, lb2t, se, sh,
                 eW2, eb2r, bv.reshape(1, _C))
    return out
